# baseline (speedup 1.0000x reference)
import functools

import jax
import jax.numpy as jnp
from jax import lax
from jax.experimental import pallas as pl
from jax.experimental.pallas import tpu as pltpu
from jax.experimental.pallas import tpu_sc as plsc

B = 16384
F = 32
NC = 2
NS = 16
NW = NC * NS
BPW = B // NW
CHUNK = 128
NCHUNK = BPW // CHUNK
GROUPS = BPW // 16


def _mf_body(uid_hbm, iid_hbm, uemb_hbm, ubias_hbm, iemb_hbm, ibias_hbm,
             out_hbm, uid_v, iid_v, urows_v, irows_v, ub_v, ib_v, out_v,
             tmp_v, sem):
    wid = lax.axis_index("s") * NC + lax.axis_index("c")
    base = wid * BPW

    pltpu.sync_copy(uid_hbm.at[pl.ds(base, BPW)], uid_v)
    pltpu.sync_copy(iid_hbm.at[pl.ds(base, BPW)], iid_v)

    copies = []
    for c in range(NCHUNK):
        sl = pl.ds(c * CHUNK, CHUNK)
        copies.append(pltpu.async_copy(uemb_hbm.at[uid_v.at[sl]], urows_v.at[sl], sem))
        copies.append(pltpu.async_copy(iemb_hbm.at[iid_v.at[sl]], irows_v.at[sl], sem))
        copies.append(pltpu.async_copy(ubias_hbm.at[uid_v.at[sl]], ub_v.at[sl], sem))
        copies.append(pltpu.async_copy(ibias_hbm.at[iid_v.at[sl]], ib_v.at[sl], sem))
    for cp in copies:
        cp.wait()

    lane16 = lax.iota(jnp.int32, 16) * 16

    def group(g, carry):
        b0 = g * 16
        for j in range(16):
            b = b0 + j
            ue, uo = plsc.unpack(urows_v[b], format=plsc.PackFormat.INTERLEAVED)
            ie, io = plsc.unpack(irows_v[b], format=plsc.PackFormat.INTERLEAVED)
            tmp_v[pl.ds(j * 16, 16)] = ue * ie + uo * io
        acc = ub_v[pl.ds(b0, 16)] + ib_v[pl.ds(b0, 16)]
        for c in range(16):
            acc = acc + plsc.load_gather(tmp_v, [lane16 + c])
        out_v[pl.ds(b0, 16)] = acc
        return carry

    lax.fori_loop(0, GROUPS, group, 0)

    pltpu.sync_copy(out_v, out_hbm.at[pl.ds(base, BPW)])


@jax.jit
def kernel(user_id, item_id, user_embedding, user_bias, item_embedding, item_bias):
    run = pl.kernel(
        _mf_body,
        out_type=jax.ShapeDtypeStruct((B,), jnp.float32),
        mesh=plsc.VectorSubcoreMesh(core_axis_name="c", subcore_axis_name="s"),
        compiler_params=pltpu.CompilerParams(
            needs_layout_passes=False, use_tc_tiling_on_sc=False),
        scratch_types=[
            pltpu.VMEM((BPW,), jnp.int32),
            pltpu.VMEM((BPW,), jnp.int32),
            pltpu.VMEM((BPW, F), jnp.bfloat16),
            pltpu.VMEM((BPW, F), jnp.bfloat16),
            pltpu.VMEM((BPW,), jnp.float32),
            pltpu.VMEM((BPW,), jnp.float32),
            pltpu.VMEM((BPW,), jnp.float32),
            pltpu.VMEM((256,), jnp.float32),
            pltpu.SemaphoreType.DMA,
        ],
    )
    return run(user_id, item_id,
               user_embedding.astype(jnp.bfloat16), user_bias.reshape(-1),
               item_embedding.astype(jnp.bfloat16), item_bias.reshape(-1))


# speedup vs baseline: 3.1307x; 3.1307x over previous
import functools

import jax
import jax.numpy as jnp
from jax import lax
from jax.experimental import pallas as pl
from jax.experimental.pallas import tpu as pltpu
from jax.experimental.pallas import tpu_sc as plsc

B = 16384
F = 32
V = 1_000_000
NC = 2
NS = 16
NW = NC * NS
BPW = B // NW
CHUNK = 128
NCHUNK = BPW // CHUNK
GROUPS = BPW // 16
VC = 65536
NVC = -(-V // VC)
FR = 8
NFR = F // FR
FLAT = NVC * VC * F


def _relayout_body(in_ref, out_ref):
    for f in range(FR):
        out_ref[pl.ds(f * VC, VC)] = in_ref[f, :]


def _relayout(tableT):
    return pl.pallas_call(
        _relayout_body,
        grid=(NVC, NFR),
        in_specs=[pl.BlockSpec((FR, VC), lambda i, t: (t, i))],
        out_specs=pl.BlockSpec((FR * VC,), lambda i, t: (i * NFR + t,)),
        out_shape=jax.ShapeDtypeStruct((FLAT,), jnp.float32),
    )(tableT)


def _mf_body(uid_hbm, iid_hbm, uflat_hbm, ubias_hbm, iflat_hbm, ibias_hbm,
             out_hbm, uid_v, iid_v, uidx_v, iidx_v, ucols_v, icols_v,
             ub_v, ib_v, out_v, sem):
    wid = lax.axis_index("s") * NC + lax.axis_index("c")
    base = wid * BPW

    pltpu.sync_copy(uid_hbm.at[pl.ds(base, BPW)], uid_v)
    pltpu.sync_copy(iid_hbm.at[pl.ds(base, BPW)], iid_v)

    for c in range(NCHUNK):
        sl = pl.ds(c * CHUNK, CHUNK)
        for j in range(CHUNK // 16):
            jsl = pl.ds(c * CHUNK + j * 16, 16)
            osl = pl.ds(j * 16, 16)
            uv = uid_v[jsl]
            iv = iid_v[jsl]
            ub_base = ((uv >> 16) << 21) + (uv & 0xFFFF)
            ib_base = ((iv >> 16) << 21) + (iv & 0xFFFF)
            for f in range(F):
                uidx_v[f, osl] = ub_base + (f << 16)
                iidx_v[f, osl] = ib_base + (f << 16)
        copies = [
            pltpu.async_copy(ubias_hbm.at[uid_v.at[sl]], ub_v.at[sl], sem),
            pltpu.async_copy(ibias_hbm.at[iid_v.at[sl]], ib_v.at[sl], sem),
        ]
        for f in range(F):
            copies.append(pltpu.async_copy(
                uflat_hbm.at[uidx_v.at[f]], ucols_v.at[f].at[sl], sem))
            copies.append(pltpu.async_copy(
                iflat_hbm.at[iidx_v.at[f]], icols_v.at[f].at[sl], sem))
        for cp in copies:
            cp.wait()

    def group(g, carry):
        i0 = g * 16
        gsl = pl.ds(i0, 16)
        acc = ub_v[gsl] + ib_v[gsl]
        for f in range(F):
            acc = acc + ucols_v[f, gsl] * icols_v[f, gsl]
        out_v[gsl] = acc
        return carry

    lax.fori_loop(0, GROUPS, group, 0)

    pltpu.sync_copy(out_v, out_hbm.at[pl.ds(base, BPW)])


@jax.jit
def kernel(user_id, item_id, user_embedding, user_bias, item_embedding, item_bias):
    run = pl.kernel(
        _mf_body,
        out_type=jax.ShapeDtypeStruct((B,), jnp.float32),
        mesh=plsc.VectorSubcoreMesh(core_axis_name="c", subcore_axis_name="s"),
        compiler_params=pltpu.CompilerParams(
            needs_layout_passes=False, use_tc_tiling_on_sc=False),
        scratch_types=[
            pltpu.VMEM((BPW,), jnp.int32),
            pltpu.VMEM((BPW,), jnp.int32),
            pltpu.VMEM((F, CHUNK), jnp.int32),
            pltpu.VMEM((F, CHUNK), jnp.int32),
            pltpu.VMEM((F, BPW), jnp.float32),
            pltpu.VMEM((F, BPW), jnp.float32),
            pltpu.VMEM((BPW,), jnp.float32),
            pltpu.VMEM((BPW,), jnp.float32),
            pltpu.VMEM((BPW,), jnp.float32),
            pltpu.SemaphoreType.DMA,
        ],
    )
    uflat = _relayout(user_embedding.T)
    iflat = _relayout(item_embedding.T)
    return run(user_id, item_id, uflat, user_bias.reshape(-1),
               iflat, item_bias.reshape(-1))


# speedup vs baseline: 3.2245x; 1.0299x over previous
import functools

import jax
import jax.numpy as jnp
from jax import lax
from jax.experimental import pallas as pl
from jax.experimental.pallas import tpu as pltpu
from jax.experimental.pallas import tpu_sc as plsc

B = 16384
F = 32
V = 1_000_000
NC = 2
NS = 16
NW = NC * NS
BPW = B // NW
CHUNK = 128
NCHUNK = BPW // CHUNK
GROUPS = BPW // 16
VC = 65536
NVC = -(-V // VC)
FR = 8
NFR = F // FR
FLAT = NVC * VC * F


def _relayout_body(in_ref, out_ref):
    for f in range(FR):
        out_ref[pl.ds(f * VC, VC)] = in_ref[f, :]


def _relayout(tableT):
    return pl.pallas_call(
        _relayout_body,
        grid=(NVC, NFR),
        in_specs=[pl.BlockSpec((FR, VC), lambda i, t: (t, i))],
        out_specs=pl.BlockSpec((FR * VC,), lambda i, t: (i * NFR + t,)),
        out_shape=jax.ShapeDtypeStruct((FLAT,), jnp.float32),
    )(tableT)


def _mf_body(uid_hbm, iid_hbm, uflat_hbm, ubias_hbm, iflat_hbm, ibias_hbm,
             out_hbm, uid_v, iid_v, uidx_v, iidx_v, ucols_v, icols_v,
             ub_v, ib_v, out_v, sem):
    wid = lax.axis_index("s") * NC + lax.axis_index("c")
    base = wid * BPW

    pltpu.sync_copy(uid_hbm.at[pl.ds(base, BPW)], uid_v)
    pltpu.sync_copy(iid_hbm.at[pl.ds(base, BPW)], iid_v)

    for c in range(NCHUNK):
        sl = pl.ds(c * CHUNK, CHUNK)
        for j in range(CHUNK // 16):
            jsl = pl.ds(c * CHUNK + j * 16, 16)
            osl = pl.ds(j * 16, 16)
            uv = uid_v[jsl]
            iv = iid_v[jsl]
            ub_base = ((uv >> 16) << 21) + (uv & 0xFFFF)
            ib_base = ((iv >> 16) << 21) + (iv & 0xFFFF)
            for f in range(F):
                uidx_v[f, osl] = ub_base + (f << 16)
                iidx_v[f, osl] = ib_base + (f << 16)
        copies = [
            pltpu.async_copy(ubias_hbm.at[0].at[uid_v.at[sl]], ub_v.at[sl], sem),
            pltpu.async_copy(ibias_hbm.at[0].at[iid_v.at[sl]], ib_v.at[sl], sem),
        ]
        for f in range(F):
            copies.append(pltpu.async_copy(
                uflat_hbm.at[uidx_v.at[f]], ucols_v.at[f].at[sl], sem))
            copies.append(pltpu.async_copy(
                iflat_hbm.at[iidx_v.at[f]], icols_v.at[f].at[sl], sem))
        for cp in copies:
            cp.wait()

    def group(g, carry):
        i0 = g * 16
        gsl = pl.ds(i0, 16)
        acc = ub_v[gsl] + ib_v[gsl]
        for f in range(F):
            acc = acc + ucols_v[f, gsl] * icols_v[f, gsl]
        out_v[gsl] = acc
        return carry

    lax.fori_loop(0, GROUPS, group, 0)

    pltpu.sync_copy(out_v, out_hbm.at[pl.ds(base, BPW)])


@jax.jit
def kernel(user_id, item_id, user_embedding, user_bias, item_embedding, item_bias):
    run = pl.kernel(
        _mf_body,
        out_type=jax.ShapeDtypeStruct((B,), jnp.float32),
        mesh=plsc.VectorSubcoreMesh(core_axis_name="c", subcore_axis_name="s"),
        compiler_params=pltpu.CompilerParams(
            needs_layout_passes=False, use_tc_tiling_on_sc=False),
        scratch_types=[
            pltpu.VMEM((BPW,), jnp.int32),
            pltpu.VMEM((BPW,), jnp.int32),
            pltpu.VMEM((F, CHUNK), jnp.int32),
            pltpu.VMEM((F, CHUNK), jnp.int32),
            pltpu.VMEM((F, BPW), jnp.float32),
            pltpu.VMEM((F, BPW), jnp.float32),
            pltpu.VMEM((BPW,), jnp.float32),
            pltpu.VMEM((BPW,), jnp.float32),
            pltpu.VMEM((BPW,), jnp.float32),
            pltpu.SemaphoreType.DMA,
        ],
    )
    uflat = _relayout(user_embedding.T)
    iflat = _relayout(item_embedding.T)
    return run(user_id, item_id, uflat, user_bias.T,
               iflat, item_bias.T)


# speedup vs baseline: 3.8671x; 1.1993x over previous
import functools

import jax
import jax.numpy as jnp
from jax import lax
from jax.experimental import pallas as pl
from jax.experimental.pallas import tpu as pltpu
from jax.experimental.pallas import tpu_sc as plsc

B = 16384
F = 32
V = 1_000_000
NC = 2
NS = 16
NW = NC * NS
BPW = B // NW
CHUNK = 128
NCHUNK = BPW // CHUNK
GROUPS = BPW // 16
VC = 65536
NVC = -(-V // VC)
FR = 8
NFR = F // FR
FLAT = NVC * VC * F


def _bias_relayout_body(in_ref, out_ref):
    out_ref[...] = in_ref[0, :]


def _relayout_body(in_ref, out_ref):
    for f in range(FR):
        out_ref[pl.ds(f * VC, VC)] = in_ref[f, :]


def _relayout(tableT):
    return pl.pallas_call(
        _relayout_body,
        grid=(NVC, NFR),
        in_specs=[pl.BlockSpec((FR, VC), lambda i, t: (t, i))],
        out_specs=pl.BlockSpec((FR * VC,), lambda i, t: (i * NFR + t,)),
        out_shape=jax.ShapeDtypeStruct((FLAT,), jnp.float32),
    )(tableT)


def _bias_relayout(biasT):
    return pl.pallas_call(
        _bias_relayout_body,
        grid=(NVC,),
        in_specs=[pl.BlockSpec((1, VC), lambda i: (0, i))],
        out_specs=pl.BlockSpec((VC,), lambda i: (i,)),
        out_shape=jax.ShapeDtypeStruct((NVC * VC,), jnp.float32),
    )(biasT)


def _mf_body(uid_hbm, iid_hbm, uflat_hbm, ubias_hbm, iflat_hbm, ibias_hbm,
             out_hbm, uid_v, iid_v, uidx_v, iidx_v, ucols_v, icols_v,
             ub_v, ib_v, out_v, sem):
    wid = lax.axis_index("s") * NC + lax.axis_index("c")
    base = wid * BPW

    pltpu.sync_copy(uid_hbm.at[pl.ds(base, BPW)], uid_v)
    pltpu.sync_copy(iid_hbm.at[pl.ds(base, BPW)], iid_v)

    for c in range(NCHUNK):
        sl = pl.ds(c * CHUNK, CHUNK)
        for j in range(CHUNK // 16):
            jsl = pl.ds(c * CHUNK + j * 16, 16)
            osl = pl.ds(j * 16, 16)
            uv = uid_v[jsl]
            iv = iid_v[jsl]
            ub_base = ((uv >> 16) << 21) + (uv & 0xFFFF)
            ib_base = ((iv >> 16) << 21) + (iv & 0xFFFF)
            for f in range(F):
                uidx_v[f, osl] = ub_base + (f << 16)
                iidx_v[f, osl] = ib_base + (f << 16)
        copies = [
            pltpu.async_copy(ubias_hbm.at[uid_v.at[sl]], ub_v.at[sl], sem),
            pltpu.async_copy(ibias_hbm.at[iid_v.at[sl]], ib_v.at[sl], sem),
        ]
        for f in range(F):
            copies.append(pltpu.async_copy(
                uflat_hbm.at[uidx_v.at[f]], ucols_v.at[f].at[sl], sem))
            copies.append(pltpu.async_copy(
                iflat_hbm.at[iidx_v.at[f]], icols_v.at[f].at[sl], sem))
        for cp in copies:
            cp.wait()

    def group(g, carry):
        i0 = g * 16
        gsl = pl.ds(i0, 16)
        acc = ub_v[gsl] + ib_v[gsl]
        for f in range(F):
            acc = acc + ucols_v[f, gsl] * icols_v[f, gsl]
        out_v[gsl] = acc
        return carry

    lax.fori_loop(0, GROUPS, group, 0)

    pltpu.sync_copy(out_v, out_hbm.at[pl.ds(base, BPW)])


@jax.jit
def kernel(user_id, item_id, user_embedding, user_bias, item_embedding, item_bias):
    run = pl.kernel(
        _mf_body,
        out_type=jax.ShapeDtypeStruct((B,), jnp.float32),
        mesh=plsc.VectorSubcoreMesh(core_axis_name="c", subcore_axis_name="s"),
        compiler_params=pltpu.CompilerParams(
            needs_layout_passes=False, use_tc_tiling_on_sc=False),
        scratch_types=[
            pltpu.VMEM((BPW,), jnp.int32),
            pltpu.VMEM((BPW,), jnp.int32),
            pltpu.VMEM((F, CHUNK), jnp.int32),
            pltpu.VMEM((F, CHUNK), jnp.int32),
            pltpu.VMEM((F, BPW), jnp.float32),
            pltpu.VMEM((F, BPW), jnp.float32),
            pltpu.VMEM((BPW,), jnp.float32),
            pltpu.VMEM((BPW,), jnp.float32),
            pltpu.VMEM((BPW,), jnp.float32),
            pltpu.SemaphoreType.DMA,
        ],
    )
    uflat = _relayout(user_embedding.T)
    iflat = _relayout(item_embedding.T)
    return run(user_id, item_id, uflat, _bias_relayout(user_bias.T),
               iflat, _bias_relayout(item_bias.T))
